# baseline (speedup 1.0000x reference)
import functools

import jax
import jax.numpy as jnp
from jax import lax
from jax.experimental import pallas as pl
from jax.experimental.pallas import tpu as pltpu
from jax.experimental.pallas import tpu_sc as plsc

D_MODEL = 64


def _make_gather(B: int, V: int, D: int):
    NW = 32
    assert B % NW == 0
    b_per_w = B // NW
    CH = 400
    NBUF = 4
    LOOKAHEAD = 2
    assert b_per_w % (CH * NBUF) == 0
    n_ch = b_per_w // CH

    mesh = plsc.VectorSubcoreMesh(core_axis_name="c", subcore_axis_name="s")

    @functools.partial(
        pl.kernel,
        mesh=mesh,
        compiler_params=pltpu.CompilerParams(use_tc_tiling_on_sc=False),
        out_type=jax.ShapeDtypeStruct((B, D), jnp.float32),
        scratch_types=[
            pltpu.VMEM((b_per_w,), jnp.int32),
            [pltpu.VMEM((CH, D), jnp.float32) for _ in range(NBUF)],
            [pltpu.SemaphoreType.DMA for _ in range(NBUF)],
            [pltpu.SemaphoreType.DMA for _ in range(NBUF)],
        ],
    )
    def k(idx_hbm, table_hbm, out_hbm, idx_v, rows_v, gsems, ssems):
        wid = lax.axis_index("s") * 2 + lax.axis_index("c")
        base = pl.multiple_of(wid * b_per_w, b_per_w)
        pltpu.sync_copy(idx_hbm.at[pl.ds(base, b_per_w)], idx_v)

        def gather_desc(c, slot):
            off = pl.multiple_of(c * CH, CH)
            return pltpu.make_async_copy(
                table_hbm.at[idx_v.at[pl.ds(off, CH)]], rows_v[slot], gsems[slot]
            )

        def store_desc(c, slot):
            off = pl.multiple_of(base + c * CH, CH)
            return pltpu.make_async_copy(
                rows_v[slot], out_hbm.at[pl.ds(off, CH)], ssems[slot]
            )

        for b in range(LOOKAHEAD):
            gather_desc(b, b).start()

        def group(g, carry):
            for b in range(NBUF):
                c = g * NBUF + b
                gather_desc(c, b).wait()
                store_desc(c, b).start()

                nxt_slot = (b + LOOKAHEAD) % NBUF
                nxt = c + LOOKAHEAD

                @pl.when(nxt < n_ch)
                def _issue_next():
                    @pl.when(nxt >= NBUF)
                    def _drain_prev():
                        store_desc(nxt - NBUF, nxt_slot).wait()

                    gather_desc(nxt, nxt_slot).start()

            return carry

        lax.fori_loop(0, n_ch // NBUF, group, 0)

        for b in range(NBUF):
            c = n_ch - NBUF + b
            store_desc(c, (n_ch - NBUF + b) % NBUF).wait()

    return k


def kernel(x, table):
    B = x.shape[0] * x.shape[1]
    idx = x.reshape(B).astype(jnp.int32)
    out = _make_gather(B, table.shape[0], table.shape[1])(idx, table)
    return out.reshape(x.shape[0], x.shape[1], table.shape[1])
